# baseline (speedup 1.0000x reference)
import functools

import jax
import jax.numpy as jnp
from jax import lax
from jax.experimental import pallas as pl
from jax.experimental.pallas import tpu as pltpu
from jax.experimental.pallas import tpu_sc as plsc

LANES = 16
NUM_WORKERS = 32
IDX_CHUNK = 128
B_CHUNK = 256


def _sc_dot_kernel(batch, embed_dim):
  b_per_w = batch // NUM_WORKERS
  n_chunks = b_per_w // B_CHUNK
  n_sub = B_CHUNK // IDX_CHUNK
  pair_dim = 2 * embed_dim

  mesh = plsc.VectorSubcoreMesh(core_axis_name="c", subcore_axis_name="s")

  @functools.partial(
      pl.kernel,
      out_type=jax.ShapeDtypeStruct((batch,), jnp.float32),
      mesh=mesh,
      compiler_params=pltpu.CompilerParams(needs_layout_passes=False),
      scratch_types=[
          pltpu.VMEM((b_per_w,), jnp.int32),
          pltpu.VMEM((b_per_w,), jnp.int32),
          pltpu.VMEM((b_per_w,), jnp.int32),
          pltpu.VMEM((b_per_w,), jnp.int32),
          pltpu.VMEM((B_CHUNK, pair_dim), jnp.float32),
          pltpu.VMEM((B_CHUNK, pair_dim), jnp.float32),
          pltpu.VMEM((b_per_w,), jnp.float32),
          pltpu.SemaphoreType.DMA,
      ],
  )
  def kern(uidx_hbm, iidx_hbm, utab_hbm, itab_hbm, out_hbm,
           uidx_v, iidx_v, uhalf_v, ihalf_v, urows_v, irows_v, out_v, sem):
    wid = lax.axis_index("s") * 2 + lax.axis_index("c")
    base = wid * b_per_w

    pltpu.sync_copy(uidx_hbm.at[pl.ds(base, b_per_w)], uidx_v)
    pltpu.sync_copy(iidx_hbm.at[pl.ds(base, b_per_w)], iidx_v)

    for g in range(b_per_w // LANES):
      sl = pl.ds(g * LANES, LANES)
      uhalf_v[sl] = lax.shift_right_logical(uidx_v[sl], 1)
      ihalf_v[sl] = lax.shift_right_logical(iidx_v[sl], 1)

    iota16 = lax.iota(jnp.int32, LANES)

    for c in range(n_chunks):
      copies = []
      for j in range(n_sub):
        off = c * B_CHUNK + j * IDX_CHUNK
        copies.append(pltpu.async_copy(
            utab_hbm.at[uhalf_v.at[pl.ds(off, IDX_CHUNK)]],
            urows_v.at[pl.ds(j * IDX_CHUNK, IDX_CHUNK)], sem))
        copies.append(pltpu.async_copy(
            itab_hbm.at[ihalf_v.at[pl.ds(off, IDX_CHUNK)]],
            irows_v.at[pl.ds(j * IDX_CHUNK, IDX_CHUNK)], sem))
      for cp in copies:
        cp.wait()

      def group_body(g, _):
        kbase = g * LANES
        rows = kbase + iota16
        ucol0 = (uidx_v[pl.ds(c * B_CHUNK + kbase, LANES)] & 1) * embed_dim
        icol0 = (iidx_v[pl.ds(c * B_CHUNK + kbase, LANES)] & 1) * embed_dim
        acc = (plsc.load_gather(urows_v, [rows, ucol0]) *
               plsc.load_gather(irows_v, [rows, icol0]))
        for d in range(1, embed_dim):
          acc = acc + (plsc.load_gather(urows_v, [rows, ucol0 + d]) *
                       plsc.load_gather(irows_v, [rows, icol0 + d]))
        out_v[pl.ds(c * B_CHUNK + kbase, LANES)] = acc
        return 0

      lax.fori_loop(0, B_CHUNK // LANES, group_body, 0)

    pltpu.sync_copy(out_v, out_hbm.at[pl.ds(base, b_per_w)])

  return kern


def kernel(user_indices, item_indices, user_weight, item_weight):
  batch = user_indices.shape[0]
  num_rows, embed_dim = user_weight.shape
  kern = _sc_dot_kernel(batch, embed_dim)
  uw2 = user_weight.reshape(num_rows // 2, 2 * embed_dim)
  iw2 = item_weight.reshape(num_rows // 2, 2 * embed_dim)
  return kern(user_indices.astype(jnp.int32), item_indices.astype(jnp.int32),
              uw2, iw2)


# speedup vs baseline: 1.4674x; 1.4674x over previous
import functools

import jax
import jax.numpy as jnp
from jax import lax
from jax.experimental import pallas as pl
from jax.experimental.pallas import tpu as pltpu
from jax.experimental.pallas import tpu_sc as plsc

LANES = 16
NUM_WORKERS = 32
B_CHUNK = 32
SLAB = 8


def _sc_dot_kernel(batch, embed_dim):
  b_per_w = batch // NUM_WORKERS
  n_chunks = b_per_w // B_CHUNK
  slab_bytes = SLAB * embed_dim * 4

  mesh = plsc.VectorSubcoreMesh(core_axis_name="c", subcore_axis_name="s")

  @functools.partial(
      pl.kernel,
      out_type=jax.ShapeDtypeStruct((batch,), jnp.float32),
      mesh=mesh,
      compiler_params=pltpu.CompilerParams(needs_layout_passes=False),
      scratch_types=[
          pltpu.VMEM((b_per_w,), jnp.int32),
          pltpu.VMEM((b_per_w,), jnp.int32),
          pltpu.VMEM((B_CHUNK * SLAB, embed_dim), jnp.float32),
          pltpu.VMEM((B_CHUNK * SLAB, embed_dim), jnp.float32),
          pltpu.VMEM((b_per_w,), jnp.float32),
          pltpu.SemaphoreType.DMA,
      ],
  )
  def kern(uidx_hbm, iidx_hbm, utab_hbm, itab_hbm, out_hbm,
           uidx_v, iidx_v, uslab_v, islab_v, out_v, sem):
    wid = lax.axis_index("s") * 2 + lax.axis_index("c")
    base = wid * b_per_w

    pltpu.sync_copy(uidx_hbm.at[pl.ds(base, b_per_w)], uidx_v)
    pltpu.sync_copy(iidx_hbm.at[pl.ds(base, b_per_w)], iidx_v)

    iota16 = lax.iota(jnp.int32, LANES)

    def chunk_body(c, _):
      for g in range(B_CHUNK // LANES):
        vu = uidx_v[pl.ds(c * B_CHUNK + g * LANES, LANES)]
        vi = iidx_v[pl.ds(c * B_CHUNK + g * LANES, LANES)]
        for l in range(LANES):
          k = g * LANES + l
          ru = vu[l]
          ri = vi[l]
          pltpu.async_copy(
              utab_hbm.at[pl.ds((ru >> 3) * SLAB, SLAB), :],
              uslab_v.at[pl.ds(k * SLAB, SLAB), :], sem)
          pltpu.async_copy(
              itab_hbm.at[pl.ds((ri >> 3) * SLAB, SLAB), :],
              islab_v.at[pl.ds(k * SLAB, SLAB), :], sem)
      pltpu.make_async_copy(utab_hbm.at[pl.ds(0, B_CHUNK * SLAB), :],
                            uslab_v, sem).wait()
      pltpu.make_async_copy(itab_hbm.at[pl.ds(0, B_CHUNK * SLAB), :],
                            islab_v, sem).wait()

      def group_body(g, _):
        kbase = g * LANES
        ruv = uidx_v[pl.ds(c * B_CHUNK + kbase, LANES)]
        riv = iidx_v[pl.ds(c * B_CHUNK + kbase, LANES)]
        urows = (kbase + iota16) * SLAB + (ruv & (SLAB - 1))
        irows = (kbase + iota16) * SLAB + (riv & (SLAB - 1))
        dcol = jnp.zeros((LANES,), jnp.int32)
        acc = (plsc.load_gather(uslab_v, [urows, dcol]) *
               plsc.load_gather(islab_v, [irows, dcol]))
        for d in range(1, embed_dim):
          dcol = jnp.full((LANES,), d, jnp.int32)
          acc = acc + (plsc.load_gather(uslab_v, [urows, dcol]) *
                       plsc.load_gather(islab_v, [irows, dcol]))
        out_v[pl.ds(c * B_CHUNK + kbase, LANES)] = acc
        return 0

      lax.fori_loop(0, B_CHUNK // LANES, group_body, 0)
      return 0

    lax.fori_loop(0, n_chunks, chunk_body, 0)

    pltpu.sync_copy(out_v, out_hbm.at[pl.ds(base, b_per_w)])

  return kern


def kernel(user_indices, item_indices, user_weight, item_weight):
  batch = user_indices.shape[0]
  num_rows, embed_dim = user_weight.shape
  kern = _sc_dot_kernel(batch, embed_dim)
  return kern(user_indices.astype(jnp.int32), item_indices.astype(jnp.int32),
              user_weight, item_weight)


# speedup vs baseline: 2.2544x; 1.5364x over previous
import functools

import jax
import jax.numpy as jnp
from jax import lax
from jax.experimental import pallas as pl
from jax.experimental.pallas import tpu as pltpu
from jax.experimental.pallas import tpu_sc as plsc

LANES = 16
NUM_WORKERS = 32
I_CHUNK = 8
U_CHUNK = 32
SLAB = 8


def _item_gather_kernel(batch, embed_dim, num_rows):
  b_per_w = batch // NUM_WORKERS
  n_chunks = b_per_w // I_CHUNK
  last_tile = num_rows // 128
  mesh = plsc.VectorSubcoreMesh(core_axis_name="c", subcore_axis_name="s")

  @functools.partial(
      pl.kernel,
      out_type=jax.ShapeDtypeStruct((batch * embed_dim,), jnp.float32),
      mesh=mesh,
      compiler_params=pltpu.CompilerParams(needs_layout_passes=False),
      scratch_types=[
          pltpu.VMEM((b_per_w + LANES,), jnp.int32),
          pltpu.VMEM((I_CHUNK, embed_dim, 128), jnp.float32),
          pltpu.VMEM((I_CHUNK * embed_dim,), jnp.float32),
          pltpu.SemaphoreType.DMA,
      ],
  )
  def kern(iidx_hbm, itab_hbm, tail_hbm, gat_hbm,
           iidx_v, slab_v, stage_v, sem):
    wid = lax.axis_index("s") * 2 + lax.axis_index("c")
    base = wid * b_per_w

    pltpu.sync_copy(iidx_hbm.at[pl.ds(base, b_per_w)],
                    iidx_v.at[pl.ds(0, b_per_w)])

    iota16 = lax.iota(jnp.int32, LANES)

    def chunk_body(c, _):
      v = iidx_v[pl.ds(c * I_CHUNK, LANES)]
      for l in range(I_CHUNK):
        ri = v[l]
        t = ri >> 7

        @pl.when(t >= last_tile)
        def _():
          pltpu.async_copy(tail_hbm, slab_v.at[l], sem)

        @pl.when(t < last_tile)
        def _():
          pltpu.async_copy(
              itab_hbm.at[:, pl.ds(t * 128, 128)], slab_v.at[l], sem)
      for l in range(I_CHUNK):
        pltpu.make_async_copy(tail_hbm, slab_v.at[l], sem).wait()

      for l in range(I_CHUNK):
        lane = jnp.full((LANES,), v[l] & 127, jnp.int32)
        slot = jnp.full((LANES,), l, jnp.int32)
        for a in range(embed_dim // LANES):
          d = a * LANES + iota16
          val = plsc.load_gather(slab_v, [slot, d, lane])
          stage_v[pl.ds(l * embed_dim + a * LANES, LANES)] = val

      pltpu.sync_copy(
          stage_v,
          gat_hbm.at[pl.ds((base + c * I_CHUNK) * embed_dim,
                           I_CHUNK * embed_dim)])
      return 0

    lax.fori_loop(0, n_chunks, chunk_body, 0)

  return kern


def _user_dot_kernel(batch, embed_dim):
  b_per_w = batch // NUM_WORKERS
  n_chunks = b_per_w // U_CHUNK
  mesh = plsc.VectorSubcoreMesh(core_axis_name="c", subcore_axis_name="s")

  @functools.partial(
      pl.kernel,
      out_type=jax.ShapeDtypeStruct((batch,), jnp.float32),
      mesh=mesh,
      compiler_params=pltpu.CompilerParams(needs_layout_passes=False),
      scratch_types=[
          pltpu.VMEM((b_per_w,), jnp.int32),
          pltpu.VMEM((U_CHUNK * SLAB, embed_dim), jnp.float32),
          pltpu.VMEM((U_CHUNK * embed_dim,), jnp.float32),
          pltpu.VMEM((b_per_w,), jnp.float32),
          pltpu.SemaphoreType.DMA,
      ],
  )
  def kern(uidx_hbm, utab_hbm, gat_hbm, out_hbm,
           uidx_v, uslab_v, igat_v, out_v, sem):
    wid = lax.axis_index("s") * 2 + lax.axis_index("c")
    base = wid * b_per_w

    pltpu.sync_copy(uidx_hbm.at[pl.ds(base, b_per_w)], uidx_v)

    iota16 = lax.iota(jnp.int32, LANES)

    def chunk_body(c, _):
      for g in range(U_CHUNK // LANES):
        vu = uidx_v[pl.ds(c * U_CHUNK + g * LANES, LANES)]
        for l in range(LANES):
          k = g * LANES + l
          ru = vu[l]
          pltpu.async_copy(
              utab_hbm.at[pl.ds((ru >> 3) * SLAB, SLAB), :],
              uslab_v.at[pl.ds(k * SLAB, SLAB), :], sem)
      pltpu.sync_copy(
          gat_hbm.at[pl.ds((base + c * U_CHUNK) * embed_dim,
                           U_CHUNK * embed_dim)], igat_v)
      pltpu.make_async_copy(utab_hbm.at[pl.ds(0, U_CHUNK * SLAB), :],
                            uslab_v, sem).wait()

      def group_body(g, _):
        kbase = g * LANES
        ruv = uidx_v[pl.ds(c * U_CHUNK + kbase, LANES)]
        urows = (kbase + iota16) * SLAB + (ruv & (SLAB - 1))
        irow0 = kbase * embed_dim + iota16 * embed_dim
        dcol = jnp.zeros((LANES,), jnp.int32)
        acc = (plsc.load_gather(uslab_v, [urows, dcol]) *
               plsc.load_gather(igat_v, [irow0]))
        for d in range(1, embed_dim):
          dcol = jnp.full((LANES,), d, jnp.int32)
          acc = acc + (plsc.load_gather(uslab_v, [urows, dcol]) *
                       plsc.load_gather(igat_v, [irow0 + d]))
        out_v[pl.ds(c * U_CHUNK + kbase, LANES)] = acc
        return 0

      lax.fori_loop(0, U_CHUNK // LANES, group_body, 0)
      return 0

    lax.fori_loop(0, n_chunks, chunk_body, 0)

    pltpu.sync_copy(out_v, out_hbm.at[pl.ds(base, b_per_w)])

  return kern


def kernel(user_indices, item_indices, user_weight, item_weight):
  batch = user_indices.shape[0]
  num_rows, embed_dim = user_weight.shape
  last_tile = num_rows // 128
  tail_rows = num_rows - last_tile * 128

  itab_t = item_weight.T
  tail = jnp.pad(item_weight[last_tile * 128:].T,
                 ((0, 0), (0, 128 - tail_rows)))

  k1 = _item_gather_kernel(batch, embed_dim, num_rows)
  igat = k1(item_indices.astype(jnp.int32), itab_t, tail)
  k2 = _user_dot_kernel(batch, embed_dim)
  return k2(user_indices.astype(jnp.int32), user_weight, igat)
